# baseline (speedup 1.0000x reference)
import functools

import jax
import jax.numpy as jnp
from jax import lax
from jax.experimental import pallas as pl
from jax.experimental.pallas import tpu as pltpu
from jax.experimental.pallas import tpu_sc as plsc

N_NODES = 10000
N_EDGES = 320000
IN_CH = 128
HID_CH = 256
OUT_CH = 128

NC = 2
NS = 16
NW = NC * NS
NPAD = 10240
RPT = NPAD // NS
K = 80
EPW = N_EDGES // NW
NCHW = EPW // K
DW = 8

EPAD = 10240
NIR = 80
KH = 64

_MESH = dict(core_axis_name="c", subcore_axis_name="s",
             num_cores=NC, num_subcores=NS)


def _writeback(acc, out0, out1, c, s):
    @pl.when(c == 0)
    def _():
        pltpu.sync_copy(acc.at[pl.ds(s * RPT, RPT)],
                        out0.at[pl.ds(s * RPT, RPT)])

    @pl.when(c == 1)
    def _():
        pltpu.sync_copy(acc.at[pl.ds(s * RPT, RPT)],
                        out1.at[pl.ds(s * RPT, RPT)])


@functools.cache
def _make_deg():
    return functools.partial(
        pl.kernel,
        out_type=[
            jax.ShapeDtypeStruct((NPAD, 128), jnp.float32),
            jax.ShapeDtypeStruct((NPAD, 128), jnp.float32),
        ],
        mesh=plsc.VectorSubcoreMesh(**_MESH),
        scratch_types=[
            pltpu.VMEM((NCHW, K), jnp.int32),
            pltpu.VMEM((K, 128), jnp.float32),
            pltpu.VMEM((K, 128), jnp.float32),
            pltpu.VMEM_SHARED((NPAD, 128), jnp.float32),
        ],
    )(_deg_body)


def _deg_body(dstw, ones_h, zero_h, deg0, deg1, dst_v, ones_v, zero_v, acc):
    c = lax.axis_index("c")
    s = lax.axis_index("s")
    wid = s * NC + c
    pltpu.sync_copy(dstw.at[wid], dst_v)
    pltpu.sync_copy(ones_h, ones_v)
    pltpu.sync_copy(zero_h, zero_v)
    for z in range(RPT // K):
        pltpu.sync_copy(zero_v, acc.at[pl.ds(s * RPT + z * K, K)])
    plsc.subcore_barrier()

    def body(j, carry):
        pltpu.sync_copy(ones_v, acc.at[dst_v.at[j]], add=True)
        return carry

    lax.fori_loop(0, NCHW, body, 0)
    plsc.subcore_barrier()
    _writeback(acc, deg0, deg1, c, s)


@functools.cache
def _make_agg():
    @functools.partial(
        pl.kernel,
        out_type=[
            jax.ShapeDtypeStruct((NPAD, IN_CH), jnp.float32),
            jax.ShapeDtypeStruct((NPAD, IN_CH), jnp.float32),
        ],
        mesh=plsc.VectorSubcoreMesh(**_MESH),
        scratch_types=[
            pltpu.VMEM((NIR, 128), jnp.int32),
            pltpu.VMEM((NIR, 128), jnp.int32),
            pltpu.VMEM((128, IN_CH), jnp.float32),
            pltpu.VMEM_SHARED((NPAD, IN_CH), jnp.float32),
        ],
    )
    def agg(tbl, srcw, dstw, zero_h, out0, out1,
            src_v, dst_v, buf_a, acc):
        c = lax.axis_index("c")
        s = lax.axis_index("s")
        wid = s * NC + c
        pltpu.sync_copy(srcw.at[wid], src_v)
        pltpu.sync_copy(dstw.at[wid], dst_v)
        pltpu.sync_copy(zero_h, buf_a)
        for z in range(RPT // 128):
            pltpu.sync_copy(buf_a, acc.at[pl.ds(s * RPT + z * 128, 128)])
        plsc.subcore_barrier()

        def body(r, carry):
            pltpu.sync_copy(tbl.at[src_v.at[r]], buf_a)
            pltpu.sync_copy(buf_a, acc.at[dst_v.at[r]], add=True)
            return carry

        lax.fori_loop(0, NIR, body, 0)
        plsc.subcore_barrier()
        _writeback(acc, out0, out1, c, s)

    return agg


_RB = 1000
_GRID = N_NODES // _RB
_ROW = lambda i: (i, 0)
_ALL = lambda i: (0, 0)


def _dinv_of(d0_ref, d1_ref):
    deg = d0_ref[:, 0] + d1_ref[:, 0] + 1.0
    return lax.rsqrt(deg)


def _tc_a_body(x_ref, w1_ref, d0_ref, d1_ref, b1_ref, u_ref, self_ref):
    dinv = _dinv_of(d0_ref, d1_ref)[:, None]
    u_ref[...] = x_ref[...] * dinv
    xw = jnp.dot(x_ref[...], w1_ref[...], preferred_element_type=jnp.float32)
    self_ref[...] = xw * (dinv * dinv) + b1_ref[...]


def _tc_a(x, W1, d0, d1, b1):
    return pl.pallas_call(
        _tc_a_body,
        grid=(_GRID,),
        in_specs=[
            pl.BlockSpec((_RB, IN_CH), _ROW),
            pl.BlockSpec((IN_CH, HID_CH), _ALL),
            pl.BlockSpec((_RB, DW), _ROW),
            pl.BlockSpec((_RB, DW), _ROW),
            pl.BlockSpec((1, HID_CH), _ALL),
        ],
        out_specs=[pl.BlockSpec((_RB, IN_CH), _ROW),
                   pl.BlockSpec((_RB, HID_CH), _ROW)],
        out_shape=[jax.ShapeDtypeStruct((N_NODES, IN_CH), jnp.float32),
                   jax.ShapeDtypeStruct((N_NODES, HID_CH), jnp.float32)],
    )(x, W1, d0, d1, b1)


def _tc_b_body(p0_ref, p1_ref, self_ref, d0_ref, d1_ref, w1_ref, w2_ref,
               b2_ref, y2_ref, self2_ref):
    dinv = _dinv_of(d0_ref, d1_ref)[:, None]
    s1 = p0_ref[...] + p1_ref[...]
    agg = jnp.dot(s1, w1_ref[...], preferred_element_type=jnp.float32)
    h = jnp.maximum(agg * dinv + self_ref[...], 0.0)
    xw2 = jnp.dot(h, w2_ref[...], preferred_element_type=jnp.float32)
    y2 = xw2 * dinv
    y2_ref[...] = y2
    self2_ref[...] = y2 * dinv + b2_ref[...]


def _tc_b(p0, p1, selft, d0, d1, W1, W2, b2):
    return pl.pallas_call(
        _tc_b_body,
        grid=(_GRID,),
        in_specs=[
            pl.BlockSpec((_RB, IN_CH), _ROW),
            pl.BlockSpec((_RB, IN_CH), _ROW),
            pl.BlockSpec((_RB, HID_CH), _ROW),
            pl.BlockSpec((_RB, DW), _ROW),
            pl.BlockSpec((_RB, DW), _ROW),
            pl.BlockSpec((IN_CH, HID_CH), _ALL),
            pl.BlockSpec((HID_CH, OUT_CH), _ALL),
            pl.BlockSpec((1, OUT_CH), _ALL),
        ],
        out_specs=[pl.BlockSpec((_RB, OUT_CH), _ROW),
                   pl.BlockSpec((_RB, OUT_CH), _ROW)],
        out_shape=[jax.ShapeDtypeStruct((N_NODES, OUT_CH), jnp.float32),
                   jax.ShapeDtypeStruct((N_NODES, OUT_CH), jnp.float32)],
    )(p0, p1, selft, d0, d1, W1, W2, b2)


def _tc_c_body(q0_ref, q1_ref, self2_ref, d0_ref, d1_ref, out_ref):
    dinv = _dinv_of(d0_ref, d1_ref)[:, None]
    out_ref[...] = (q0_ref[...] + q1_ref[...]) * dinv + self2_ref[...]


def _tc_c(q0, q1, self2, d0, d1):
    return pl.pallas_call(
        _tc_c_body,
        grid=(_GRID,),
        in_specs=[
            pl.BlockSpec((_RB, OUT_CH), _ROW),
            pl.BlockSpec((_RB, OUT_CH), _ROW),
            pl.BlockSpec((_RB, OUT_CH), _ROW),
            pl.BlockSpec((_RB, DW), _ROW),
            pl.BlockSpec((_RB, DW), _ROW),
        ],
        out_specs=pl.BlockSpec((_RB, OUT_CH), _ROW),
        out_shape=jax.ShapeDtypeStruct((N_NODES, OUT_CH), jnp.float32),
    )(q0, q1, self2, d0, d1)


def kernel(x, edge_index, W1, b1, W2, b2):
    ei = edge_index.astype(jnp.int32)
    npd = EPAD - EPW
    srcw = jnp.pad(ei[0].reshape(NW, EPW), ((0, 0), (0, npd))
                   ).reshape(NW, NIR, 128)
    dstw = jnp.pad(ei[1].reshape(NW, EPW), ((0, 0), (0, npd)),
                   constant_values=N_NODES).reshape(NW, NIR, 128)
    dstw_deg = ei[1].reshape(NW, NCHW, K)
    ones_h = jnp.ones((K, 128), jnp.float32)
    zdeg_h = jnp.zeros((K, 128), jnp.float32)
    zf32_h = jnp.zeros((128, IN_CH), jnp.float32)

    agg = _make_agg()
    deg0, deg1 = _make_deg()(dstw_deg, ones_h, zdeg_h)
    d0 = deg0[:N_NODES, :DW]
    d1 = deg1[:N_NODES, :DW]
    u, selft = _tc_a(x, W1, d0, d1, b1.reshape(1, HID_CH))
    p0, p1 = agg(u, srcw, dstw, zf32_h)
    y2, self2 = _tc_b(p0[:N_NODES], p1[:N_NODES], selft, d0, d1, W1, W2,
                      b2.reshape(1, OUT_CH))
    q0, q1 = agg(y2, srcw, dstw, zf32_h)
    return _tc_c(q0[:N_NODES], q1[:N_NODES], self2, d0, d1)


# speedup vs baseline: 2.0414x; 2.0414x over previous
import functools

import jax
import jax.numpy as jnp
from jax import lax
from jax.experimental import pallas as pl
from jax.experimental.pallas import tpu as pltpu
from jax.experimental.pallas import tpu_sc as plsc

N_NODES = 10000
N_EDGES = 320000
IN_CH = 128
HID_CH = 256
OUT_CH = 128

NC = 2
NS = 16
NW = NC * NS
NPAD = 10240
RPT = NPAD // NS
K = 80
EPW = N_EDGES // NW
NCHW = EPW // K
DW = 8

EPAD = 10240
NIR = 80
KH = 64

_MESH = dict(core_axis_name="c", subcore_axis_name="s",
             num_cores=NC, num_subcores=NS)


def _writeback(acc, out0, out1, c, s):
    @pl.when(c == 0)
    def _():
        pltpu.sync_copy(acc.at[pl.ds(s * RPT, RPT)],
                        out0.at[pl.ds(s * RPT, RPT)])

    @pl.when(c == 1)
    def _():
        pltpu.sync_copy(acc.at[pl.ds(s * RPT, RPT)],
                        out1.at[pl.ds(s * RPT, RPT)])


@functools.cache
def _make_deg():
    return functools.partial(
        pl.kernel,
        out_type=[
            jax.ShapeDtypeStruct((NPAD, 128), jnp.float32),
            jax.ShapeDtypeStruct((NPAD, 128), jnp.float32),
        ],
        mesh=plsc.VectorSubcoreMesh(**_MESH),
        scratch_types=[
            pltpu.VMEM((NCHW, K), jnp.int32),
            pltpu.VMEM((K, 128), jnp.float32),
            pltpu.VMEM((K, 128), jnp.float32),
            pltpu.VMEM_SHARED((NPAD, 128), jnp.float32),
        ],
    )(_deg_body)


def _deg_body(dstw, ones_h, zero_h, deg0, deg1, dst_v, ones_v, zero_v, acc):
    c = lax.axis_index("c")
    s = lax.axis_index("s")
    wid = s * NC + c
    pltpu.sync_copy(dstw.at[wid], dst_v)
    pltpu.sync_copy(ones_h, ones_v)
    pltpu.sync_copy(zero_h, zero_v)
    for z in range(RPT // K):
        pltpu.sync_copy(zero_v, acc.at[pl.ds(s * RPT + z * K, K)])
    plsc.subcore_barrier()

    def body(j, carry):
        pltpu.sync_copy(ones_v, acc.at[dst_v.at[j]], add=True)
        return carry

    lax.fori_loop(0, NCHW, body, 0)
    plsc.subcore_barrier()
    _writeback(acc, deg0, deg1, c, s)


@functools.cache
def _make_agg():
    @functools.partial(
        pl.kernel,
        out_type=[
            jax.ShapeDtypeStruct((NPAD, IN_CH), jnp.float32),
            jax.ShapeDtypeStruct((NPAD, IN_CH), jnp.float32),
        ],
        mesh=plsc.VectorSubcoreMesh(**_MESH),
        scratch_types=[
            pltpu.VMEM((NIR, 128), jnp.int32),
            pltpu.VMEM((NIR, 128), jnp.int32),
            pltpu.VMEM((128, IN_CH), jnp.float32),
            pltpu.VMEM_SHARED((NPAD, IN_CH), jnp.float32),
        ],
    )
    def agg(tbl, srcw, dstw, zero_h, out0, out1,
            src_v, dst_v, buf_a, acc):
        c = lax.axis_index("c")
        s = lax.axis_index("s")
        wid = s * NC + c
        pltpu.sync_copy(srcw.at[wid], src_v)
        pltpu.sync_copy(dstw.at[wid], dst_v)
        pltpu.sync_copy(zero_h, buf_a)
        for z in range(RPT // 128):
            pltpu.sync_copy(buf_a, acc.at[pl.ds(s * RPT + z * 128, 128)])
        plsc.subcore_barrier()

        def body(r, carry):
            pltpu.sync_copy(tbl.at[src_v.at[r]], buf_a)
            pltpu.sync_copy(buf_a, acc.at[dst_v.at[r]], add=True)
            return carry

        lax.fori_loop(0, NIR, body, 0)
        plsc.subcore_barrier()
        _writeback(acc, out0, out1, c, s)

    return agg


_RB = 1000
_GRID = N_NODES // _RB
_ROW = lambda i: (i, 0)
_ALL = lambda i: (0, 0)


def _dinv_of(d0_ref, d1_ref):
    deg = d0_ref[:, 0] + d1_ref[:, 0] + 1.0
    return lax.rsqrt(deg)


def _tc_a_body(x_ref, w1_ref, d0_ref, d1_ref, b1_ref, u_ref, self_ref):
    dinv = _dinv_of(d0_ref, d1_ref)[:, None]
    u_ref[...] = x_ref[...] * dinv
    xw = jnp.dot(x_ref[...], w1_ref[...], preferred_element_type=jnp.float32)
    self_ref[...] = xw * (dinv * dinv) + b1_ref[...]


def _tc_a(x, W1, d0, d1, b1):
    return pl.pallas_call(
        _tc_a_body,
        grid=(_GRID,),
        in_specs=[
            pl.BlockSpec((_RB, IN_CH), _ROW),
            pl.BlockSpec((IN_CH, HID_CH), _ALL),
            pl.BlockSpec((_RB, DW), _ROW),
            pl.BlockSpec((_RB, DW), _ROW),
            pl.BlockSpec((1, HID_CH), _ALL),
        ],
        out_specs=[pl.BlockSpec((_RB, IN_CH), _ROW),
                   pl.BlockSpec((_RB, HID_CH), _ROW)],
        out_shape=[jax.ShapeDtypeStruct((N_NODES, IN_CH), jnp.float32),
                   jax.ShapeDtypeStruct((N_NODES, HID_CH), jnp.float32)],
    )(x, W1, d0, d1, b1)


def _tc_b_body(p0_ref, p1_ref, self_ref, d0_ref, d1_ref, w1_ref, w2_ref,
               b2_ref, y2_ref, self2_ref):
    dinv = _dinv_of(d0_ref, d1_ref)[:, None]
    s1 = p0_ref[...] + p1_ref[...]
    agg = jnp.dot(s1, w1_ref[...], preferred_element_type=jnp.float32)
    h = jnp.maximum(agg * dinv + self_ref[...], 0.0)
    xw2 = jnp.dot(h, w2_ref[...], preferred_element_type=jnp.float32)
    y2 = xw2 * dinv
    y2_ref[...] = y2
    self2_ref[...] = y2 * dinv + b2_ref[...]


def _tc_b(p0, p1, selft, d0, d1, W1, W2, b2):
    return pl.pallas_call(
        _tc_b_body,
        grid=(_GRID,),
        in_specs=[
            pl.BlockSpec((_RB, IN_CH), _ROW),
            pl.BlockSpec((_RB, IN_CH), _ROW),
            pl.BlockSpec((_RB, HID_CH), _ROW),
            pl.BlockSpec((_RB, DW), _ROW),
            pl.BlockSpec((_RB, DW), _ROW),
            pl.BlockSpec((IN_CH, HID_CH), _ALL),
            pl.BlockSpec((HID_CH, OUT_CH), _ALL),
            pl.BlockSpec((1, OUT_CH), _ALL),
        ],
        out_specs=[pl.BlockSpec((_RB, OUT_CH), _ROW),
                   pl.BlockSpec((_RB, OUT_CH), _ROW)],
        out_shape=[jax.ShapeDtypeStruct((N_NODES, OUT_CH), jnp.float32),
                   jax.ShapeDtypeStruct((N_NODES, OUT_CH), jnp.float32)],
    )(p0, p1, selft, d0, d1, W1, W2, b2)


def _tc_c_body(q0_ref, q1_ref, self2_ref, d0_ref, d1_ref, out_ref):
    dinv = _dinv_of(d0_ref, d1_ref)[:, None]
    out_ref[...] = (q0_ref[...] + q1_ref[...]) * dinv + self2_ref[...]


def _tc_c(q0, q1, self2, d0, d1):
    return pl.pallas_call(
        _tc_c_body,
        grid=(_GRID,),
        in_specs=[
            pl.BlockSpec((_RB, OUT_CH), _ROW),
            pl.BlockSpec((_RB, OUT_CH), _ROW),
            pl.BlockSpec((_RB, OUT_CH), _ROW),
            pl.BlockSpec((_RB, DW), _ROW),
            pl.BlockSpec((_RB, DW), _ROW),
        ],
        out_specs=pl.BlockSpec((_RB, OUT_CH), _ROW),
        out_shape=jax.ShapeDtypeStruct((N_NODES, OUT_CH), jnp.float32),
    )(q0, q1, self2, d0, d1)


def kernel(x, edge_index, W1, b1, W2, b2):
    ei = edge_index.astype(jnp.int32)
    npd = EPAD - EPW
    jsrc = jnp.broadcast_to((jnp.arange(npd, dtype=jnp.int32) * 41)
                            % N_NODES, (NW, npd))
    jdst = jnp.broadcast_to(N_NODES + jnp.arange(npd, dtype=jnp.int32)
                            % (NPAD - N_NODES), (NW, npd))
    srcw = jnp.concatenate([ei[0].reshape(NW, EPW), jsrc],
                           axis=1).reshape(NW, NIR, 128)
    dstw = jnp.concatenate([ei[1].reshape(NW, EPW), jdst],
                           axis=1).reshape(NW, NIR, 128)
    dstw_deg = ei[1].reshape(NW, NCHW, K)
    ones_h = jnp.ones((K, 128), jnp.float32)
    zdeg_h = jnp.zeros((K, 128), jnp.float32)
    zf32_h = jnp.zeros((128, IN_CH), jnp.float32)

    agg = _make_agg()
    deg0, deg1 = _make_deg()(dstw_deg, ones_h, zdeg_h)
    d0 = deg0[:N_NODES, :DW]
    d1 = deg1[:N_NODES, :DW]
    u, selft = _tc_a(x, W1, d0, d1, b1.reshape(1, HID_CH))
    p0, p1 = agg(u, srcw, dstw, zf32_h)
    y2, self2 = _tc_b(p0[:N_NODES], p1[:N_NODES], selft, d0, d1, W1, W2,
                      b2.reshape(1, OUT_CH))
    q0, q1 = agg(y2, srcw, dstw, zf32_h)
    return _tc_c(q0[:N_NODES], q1[:N_NODES], self2, d0, d1)


# speedup vs baseline: 2.0468x; 1.0026x over previous
import functools

import jax
import jax.numpy as jnp
from jax import lax
from jax.experimental import pallas as pl
from jax.experimental.pallas import tpu as pltpu
from jax.experimental.pallas import tpu_sc as plsc

N_NODES = 10000
N_EDGES = 320000
IN_CH = 128
HID_CH = 256
OUT_CH = 128

NC = 2
NS = 16
NW = NC * NS
NPAD = 10240
RPT = NPAD // NS
K = 80
EPW = N_EDGES // NW
NCHW = EPW // K
DW = 8

EPAD = 10240
NIR = 80
KH = 64

_MESH = dict(core_axis_name="c", subcore_axis_name="s",
             num_cores=NC, num_subcores=NS)


def _writeback(acc, out0, out1, c, s):
    @pl.when(c == 0)
    def _():
        pltpu.sync_copy(acc.at[pl.ds(s * RPT, RPT)],
                        out0.at[pl.ds(s * RPT, RPT)])

    @pl.when(c == 1)
    def _():
        pltpu.sync_copy(acc.at[pl.ds(s * RPT, RPT)],
                        out1.at[pl.ds(s * RPT, RPT)])


@functools.cache
def _make_deg():
    return functools.partial(
        pl.kernel,
        out_type=[
            jax.ShapeDtypeStruct((NPAD, 128), jnp.float32),
            jax.ShapeDtypeStruct((NPAD, 128), jnp.float32),
        ],
        mesh=plsc.VectorSubcoreMesh(**_MESH),
        scratch_types=[
            pltpu.VMEM((NCHW, K), jnp.int32),
            pltpu.VMEM((K, 128), jnp.float32),
            pltpu.VMEM((K, 128), jnp.float32),
            pltpu.VMEM_SHARED((NPAD, 128), jnp.float32),
        ],
    )(_deg_body)


def _deg_body(dstw, ones_h, zero_h, deg0, deg1, dst_v, ones_v, zero_v, acc):
    c = lax.axis_index("c")
    s = lax.axis_index("s")
    wid = s * NC + c
    pltpu.sync_copy(dstw.at[wid], dst_v)
    pltpu.sync_copy(ones_h, ones_v)
    pltpu.sync_copy(zero_h, zero_v)
    for z in range(RPT // K):
        pltpu.sync_copy(zero_v, acc.at[pl.ds(s * RPT + z * K, K)])
    plsc.subcore_barrier()

    def body(j, carry):
        pltpu.sync_copy(ones_v, acc.at[dst_v.at[j]], add=True)
        return carry

    lax.fori_loop(0, NCHW, body, 0)
    plsc.subcore_barrier()
    _writeback(acc, deg0, deg1, c, s)


@functools.cache
def _make_agg():
    @functools.partial(
        pl.kernel,
        out_type=[
            jax.ShapeDtypeStruct((NPAD, IN_CH), jnp.float32),
            jax.ShapeDtypeStruct((NPAD, IN_CH), jnp.float32),
        ],
        mesh=plsc.VectorSubcoreMesh(**_MESH),
        scratch_types=[
            pltpu.VMEM((NIR, 128), jnp.int32),
            pltpu.VMEM((NIR, 128), jnp.int32),
            pltpu.VMEM((KH, IN_CH), jnp.float32),
            pltpu.VMEM((KH, IN_CH), jnp.float32),
            pltpu.SemaphoreType.DMA,
            pltpu.SemaphoreType.DMA,
            pltpu.VMEM_SHARED((NPAD, IN_CH), jnp.float32),
        ],
    )
    def agg(tbl, srcw, dstw, zero_h, out0, out1,
            src_v, dst_v, buf_a, buf_b, sem_a, sem_b, acc):
        c = lax.axis_index("c")
        s = lax.axis_index("s")
        wid = s * NC + c
        pltpu.sync_copy(srcw.at[wid], src_v)
        pltpu.sync_copy(dstw.at[wid], dst_v)
        pltpu.sync_copy(zero_h, buf_a)
        for z in range(RPT // KH):
            pltpu.sync_copy(buf_a, acc.at[pl.ds(s * RPT + z * KH, KH)])
        plsc.subcore_barrier()

        def ga(r):
            return pltpu.make_async_copy(
                tbl.at[src_v.at[r, pl.ds(0, KH)]], buf_a, sem_a)

        def gb(r):
            return pltpu.make_async_copy(
                tbl.at[src_v.at[r, pl.ds(KH, KH)]], buf_b, sem_b)

        ga(0).start()

        def body(r, carry):
            ga(r).wait()
            gb(r).start()
            pltpu.sync_copy(buf_a, acc.at[dst_v.at[r, pl.ds(0, KH)]],
                            add=True)
            gb(r).wait()
            ga(r + 1).start()
            pltpu.sync_copy(buf_b, acc.at[dst_v.at[r, pl.ds(KH, KH)]],
                            add=True)
            return carry

        lax.fori_loop(0, NIR - 1, body, 0)
        rl = NIR - 1
        ga(rl).wait()
        gb(rl).start()
        pltpu.sync_copy(buf_a, acc.at[dst_v.at[rl, pl.ds(0, KH)]], add=True)
        gb(rl).wait()
        pltpu.sync_copy(buf_b, acc.at[dst_v.at[rl, pl.ds(KH, KH)]], add=True)
        plsc.subcore_barrier()
        _writeback(acc, out0, out1, c, s)

    return agg


_RB = 1000
_GRID = N_NODES // _RB
_ROW = lambda i: (i, 0)
_ALL = lambda i: (0, 0)


def _dinv_of(d0_ref, d1_ref):
    deg = d0_ref[:, 0] + d1_ref[:, 0] + 1.0
    return lax.rsqrt(deg)


def _tc_a_body(x_ref, w1_ref, d0_ref, d1_ref, b1_ref, u_ref, self_ref):
    dinv = _dinv_of(d0_ref, d1_ref)[:, None]
    u_ref[...] = x_ref[...] * dinv
    xw = jnp.dot(x_ref[...], w1_ref[...], preferred_element_type=jnp.float32)
    self_ref[...] = xw * (dinv * dinv) + b1_ref[...]


def _tc_a(x, W1, d0, d1, b1):
    return pl.pallas_call(
        _tc_a_body,
        grid=(_GRID,),
        in_specs=[
            pl.BlockSpec((_RB, IN_CH), _ROW),
            pl.BlockSpec((IN_CH, HID_CH), _ALL),
            pl.BlockSpec((_RB, DW), _ROW),
            pl.BlockSpec((_RB, DW), _ROW),
            pl.BlockSpec((1, HID_CH), _ALL),
        ],
        out_specs=[pl.BlockSpec((_RB, IN_CH), _ROW),
                   pl.BlockSpec((_RB, HID_CH), _ROW)],
        out_shape=[jax.ShapeDtypeStruct((N_NODES, IN_CH), jnp.float32),
                   jax.ShapeDtypeStruct((N_NODES, HID_CH), jnp.float32)],
    )(x, W1, d0, d1, b1)


def _tc_b_body(p0_ref, p1_ref, self_ref, d0_ref, d1_ref, w1_ref, w2_ref,
               b2_ref, y2_ref, self2_ref):
    dinv = _dinv_of(d0_ref, d1_ref)[:, None]
    s1 = p0_ref[...] + p1_ref[...]
    agg = jnp.dot(s1, w1_ref[...], preferred_element_type=jnp.float32)
    h = jnp.maximum(agg * dinv + self_ref[...], 0.0)
    xw2 = jnp.dot(h, w2_ref[...], preferred_element_type=jnp.float32)
    y2 = xw2 * dinv
    y2_ref[...] = y2
    self2_ref[...] = y2 * dinv + b2_ref[...]


def _tc_b(p0, p1, selft, d0, d1, W1, W2, b2):
    return pl.pallas_call(
        _tc_b_body,
        grid=(_GRID,),
        in_specs=[
            pl.BlockSpec((_RB, IN_CH), _ROW),
            pl.BlockSpec((_RB, IN_CH), _ROW),
            pl.BlockSpec((_RB, HID_CH), _ROW),
            pl.BlockSpec((_RB, DW), _ROW),
            pl.BlockSpec((_RB, DW), _ROW),
            pl.BlockSpec((IN_CH, HID_CH), _ALL),
            pl.BlockSpec((HID_CH, OUT_CH), _ALL),
            pl.BlockSpec((1, OUT_CH), _ALL),
        ],
        out_specs=[pl.BlockSpec((_RB, OUT_CH), _ROW),
                   pl.BlockSpec((_RB, OUT_CH), _ROW)],
        out_shape=[jax.ShapeDtypeStruct((N_NODES, OUT_CH), jnp.float32),
                   jax.ShapeDtypeStruct((N_NODES, OUT_CH), jnp.float32)],
    )(p0, p1, selft, d0, d1, W1, W2, b2)


def _tc_c_body(q0_ref, q1_ref, self2_ref, d0_ref, d1_ref, out_ref):
    dinv = _dinv_of(d0_ref, d1_ref)[:, None]
    out_ref[...] = (q0_ref[...] + q1_ref[...]) * dinv + self2_ref[...]


def _tc_c(q0, q1, self2, d0, d1):
    return pl.pallas_call(
        _tc_c_body,
        grid=(_GRID,),
        in_specs=[
            pl.BlockSpec((_RB, OUT_CH), _ROW),
            pl.BlockSpec((_RB, OUT_CH), _ROW),
            pl.BlockSpec((_RB, OUT_CH), _ROW),
            pl.BlockSpec((_RB, DW), _ROW),
            pl.BlockSpec((_RB, DW), _ROW),
        ],
        out_specs=pl.BlockSpec((_RB, OUT_CH), _ROW),
        out_shape=jax.ShapeDtypeStruct((N_NODES, OUT_CH), jnp.float32),
    )(q0, q1, self2, d0, d1)


def kernel(x, edge_index, W1, b1, W2, b2):
    ei = edge_index.astype(jnp.int32)
    npd = EPAD - EPW
    jsrc = jnp.broadcast_to((jnp.arange(npd, dtype=jnp.int32) * 41)
                            % N_NODES, (NW, npd))
    jdst = jnp.broadcast_to(N_NODES + jnp.arange(npd, dtype=jnp.int32)
                            % (NPAD - N_NODES), (NW, npd))
    srcw = jnp.concatenate([ei[0].reshape(NW, EPW), jsrc],
                           axis=1).reshape(NW, NIR, 128)
    dstw = jnp.concatenate([ei[1].reshape(NW, EPW), jdst],
                           axis=1).reshape(NW, NIR, 128)
    dstw_deg = ei[1].reshape(NW, NCHW, K)
    ones_h = jnp.ones((K, 128), jnp.float32)
    zdeg_h = jnp.zeros((K, 128), jnp.float32)
    zf32_h = jnp.zeros((KH, IN_CH), jnp.float32)

    agg = _make_agg()
    deg0, deg1 = _make_deg()(dstw_deg, ones_h, zdeg_h)
    d0 = deg0[:N_NODES, :DW]
    d1 = deg1[:N_NODES, :DW]
    u, selft = _tc_a(x, W1, d0, d1, b1.reshape(1, HID_CH))
    p0, p1 = agg(u, srcw, dstw, zf32_h)
    y2, self2 = _tc_b(p0[:N_NODES], p1[:N_NODES], selft, d0, d1, W1, W2,
                      b2.reshape(1, OUT_CH))
    q0, q1 = agg(y2, srcw, dstw, zf32_h)
    return _tc_c(q0[:N_NODES], q1[:N_NODES], self2, d0, d1)
